# baseline (speedup 1.0000x reference)
import functools

import jax
import jax.numpy as jnp
from jax import lax
from jax.experimental import pallas as pl
from jax.experimental.pallas import tpu as pltpu
from jax.experimental.pallas import tpu_sc as plsc

EMBED_DIM = 64
PAD_DIM = 128
CHUNK = 256
NBUF = 2
VLEN = 16


@functools.cache
def _make_kernel(n_idx: int):
    info = plsc.get_sparse_core_info()
    num_cores = info.num_cores
    num_workers = info.num_cores * info.num_subcores
    b_per_w = n_idx // num_workers
    assert n_idx % num_workers == 0 and b_per_w % CHUNK == 0
    n_chunks = b_per_w // CHUNK
    n_groups = n_chunks // NBUF
    assert n_chunks % NBUF == 0 and n_groups >= 3

    mesh = plsc.VectorSubcoreMesh(core_axis_name="c", subcore_axis_name="s")

    @functools.partial(
        pl.kernel,
        mesh=mesh,
        out_type=jax.ShapeDtypeStruct((n_idx // 2, PAD_DIM), jnp.float32),
        scratch_types=[
            pltpu.VMEM((b_per_w,), jnp.int32),
            pltpu.VMEM((NBUF, CHUNK, PAD_DIM), jnp.float32),
            pltpu.SemaphoreType.DMA,
            pltpu.SemaphoreType.DMA,
        ],
    )
    def emb_kernel(idx_hbm, table_hbm, out_hbm, idx_v, rows_v, gsem, osem):
        wid = lax.axis_index("s") * num_cores + lax.axis_index("c")
        base = wid * b_per_w
        pltpu.sync_copy(idx_hbm.at[pl.ds(base, b_per_w)], idx_v)

        def start_gather(i, b):
            off = pl.multiple_of(i * CHUNK, CHUNK)
            pltpu.async_copy(
                table_hbm.at[idx_v.at[pl.ds(off, CHUNK)]], rows_v.at[b], gsem)

        def wait_gather(b):
            pltpu.make_async_copy(
                table_hbm.at[idx_v.at[pl.ds(0, CHUNK)]], rows_v.at[b], gsem
            ).wait()

        def compact(b):
            @pl.loop(0, CHUNK // 2)
            def _(k):
                r0 = 2 * k
                r1 = r0 + 1
                for t in range(EMBED_DIM // VLEN):
                    sl = pl.ds(VLEN * t, VLEN)
                    rows_v[b, k, sl] = rows_v[b, r0, sl]
                for t in range(EMBED_DIM // VLEN):
                    rows_v[b, k, pl.ds(EMBED_DIM + VLEN * t, VLEN)] = (
                        rows_v[b, r1, pl.ds(VLEN * t, VLEN)])

        def start_out(i, b):
            off = pl.multiple_of((base + i * CHUNK) // 2, CHUNK // 2)
            pltpu.async_copy(
                rows_v.at[b, pl.ds(0, CHUNK // 2)],
                out_hbm.at[pl.ds(off, CHUNK // 2)], osem)

        def wait_out(b):
            pltpu.make_async_copy(
                rows_v.at[b, pl.ds(0, CHUNK // 2)],
                out_hbm.at[pl.ds(0, CHUNK // 2)], osem
            ).wait()

        def visit(i, b, retire_prev=True, start_next=True):
            wait_gather(b)
            if retire_prev:
                wait_out((b - 1) % NBUF)
            if start_next:
                start_gather(i - 1 + NBUF, (b - 1) % NBUF)
            compact(b)
            start_out(i, b)

        for b in range(NBUF):
            start_gather(b, b)

        for b in range(NBUF):
            visit(b, b, retire_prev=b >= 1, start_next=b >= 1)

        @pl.loop(1, n_groups - 1)
        def _(t):
            for b in range(NBUF):
                visit(t * NBUF + b, b, start_next=True)

        for b in range(NBUF):
            i = (n_groups - 1) * NBUF + b
            visit(i, b, start_next=(i - 1 + NBUF) < n_chunks)

        wait_out((n_chunks - 1) % NBUF)

    return emb_kernel


@jax.jit
def kernel(input_id, table):
    batch, seq_len = input_id.shape
    flat_idx = input_id.reshape(batch * seq_len)
    padded = jnp.pad(table, ((0, 0), (0, PAD_DIM - EMBED_DIM)))
    out = _make_kernel(batch * seq_len)(flat_idx, padded)
    return out.reshape(batch, seq_len, EMBED_DIM)


# speedup vs baseline: 1.2283x; 1.2283x over previous
import functools

import jax
import jax.numpy as jnp
from jax import lax
from jax.experimental import pallas as pl
from jax.experimental.pallas import tpu as pltpu
from jax.experimental.pallas import tpu_sc as plsc

EMBED_DIM = 64
PAD_DIM = 128
CHUNK = 256
NBUF = 2
VLEN = 16


@functools.cache
def _make_kernel(n_idx: int):
    info = plsc.get_sparse_core_info()
    num_cores = info.num_cores
    num_workers = info.num_cores * info.num_subcores
    b_per_w = n_idx // num_workers
    assert n_idx % num_workers == 0 and b_per_w % CHUNK == 0
    n_chunks = b_per_w // CHUNK
    n_groups = n_chunks // NBUF
    assert n_chunks % NBUF == 0 and n_groups >= 3

    mesh = plsc.VectorSubcoreMesh(core_axis_name="c", subcore_axis_name="s")

    @functools.partial(
        pl.kernel,
        mesh=mesh,
        out_type=jax.ShapeDtypeStruct((n_idx // 2, PAD_DIM), jnp.float32),
        scratch_types=[
            pltpu.VMEM((b_per_w,), jnp.int32),
            pltpu.VMEM((NBUF, CHUNK, PAD_DIM), jnp.float32),
            pltpu.VMEM((NBUF, CHUNK // 2, PAD_DIM), jnp.float32),
            pltpu.SemaphoreType.DMA,
            pltpu.SemaphoreType.DMA,
        ],
    )
    def emb_kernel(idx_hbm, table_hbm, out_hbm, idx_v, rows_v, cmp_v, gsem,
                   osem):
        wid = lax.axis_index("s") * num_cores + lax.axis_index("c")
        base = wid * b_per_w
        pltpu.sync_copy(idx_hbm.at[pl.ds(base, b_per_w)], idx_v)

        def start_gather(i, b):
            off = pl.multiple_of(i * CHUNK, CHUNK)
            pltpu.async_copy(
                table_hbm.at[idx_v.at[pl.ds(off, CHUNK)]], rows_v.at[b], gsem)

        def wait_gather(b):
            pltpu.make_async_copy(
                table_hbm.at[idx_v.at[pl.ds(0, CHUNK)]], rows_v.at[b], gsem
            ).wait()

        def compact(b):
            @plsc.parallel_loop(0, CHUNK // 2, unroll=4)
            def _(k):
                r0 = 2 * k
                r1 = r0 + 1
                for t in range(EMBED_DIM // VLEN):
                    sl = pl.ds(VLEN * t, VLEN)
                    cmp_v[b, k, sl] = rows_v[b, r0, sl]
                for t in range(EMBED_DIM // VLEN):
                    cmp_v[b, k, pl.ds(EMBED_DIM + VLEN * t, VLEN)] = (
                        rows_v[b, r1, pl.ds(VLEN * t, VLEN)])

        def start_out(i, b):
            off = pl.multiple_of((base + i * CHUNK) // 2, CHUNK // 2)
            pltpu.async_copy(
                cmp_v.at[b], out_hbm.at[pl.ds(off, CHUNK // 2)], osem)

        def wait_out(b):
            pltpu.make_async_copy(
                cmp_v.at[b], out_hbm.at[pl.ds(0, CHUNK // 2)], osem
            ).wait()

        def visit(i, b, retire_prev=True, start_next=True):
            wait_gather(b)
            if retire_prev:
                wait_out((b - 1) % NBUF)
            if start_next:
                start_gather(i - 1 + NBUF, (b - 1) % NBUF)
            compact(b)
            start_out(i, b)

        for b in range(NBUF):
            start_gather(b, b)

        for b in range(NBUF):
            visit(b, b, retire_prev=b >= 1, start_next=b >= 1)

        @pl.loop(1, n_groups - 1)
        def _(t):
            for b in range(NBUF):
                visit(t * NBUF + b, b, start_next=True)

        for b in range(NBUF):
            i = (n_groups - 1) * NBUF + b
            visit(i, b, start_next=(i - 1 + NBUF) < n_chunks)

        wait_out((n_chunks - 1) % NBUF)

    return emb_kernel


@jax.jit
def kernel(input_id, table):
    batch, seq_len = input_id.shape
    flat_idx = input_id.reshape(batch * seq_len)
    padded = jnp.pad(table, ((0, 0), (0, PAD_DIM - EMBED_DIM)))
    out = _make_kernel(batch * seq_len)(flat_idx, padded)
    return out.reshape(batch, seq_len, EMBED_DIM)
